# speedup vs baseline: 1.0041x; 1.0041x over previous
import functools

import jax
import jax.numpy as jnp
from jax import lax
from jax.experimental import pallas as pl

N_USERS = 5000
N_ITEMS = 5000
N_NODES = N_USERS + N_ITEMS
E_EDGES = 320000
D = 128
N_INTENTS = 128
N_LAYERS = 2

ROW_BLK = 1000


def _intent_body(x_ref, w_ref, o_ref):
    x = x_ref[...]
    w = w_ref[...]
    logits = jnp.dot(x, w, preferred_element_type=jnp.float32)
    m = jnp.max(logits, axis=1, keepdims=True)
    p = jnp.exp(logits - m)
    s = jnp.sum(p, axis=1, keepdims=True)
    p = p / s
    o_ref[...] = lax.dot_general(p, w, (((1,), (1,)), ((), ())),
                                 preferred_element_type=jnp.float32)


def _intent(x, w):
    n = x.shape[0]
    grid = n // ROW_BLK
    return pl.pallas_call(
        _intent_body,
        grid=(grid,),
        in_specs=[
            pl.BlockSpec((ROW_BLK, D), lambda i: (i, 0)),
            pl.BlockSpec((D, N_INTENTS), lambda i: (0, 0)),
        ],
        out_specs=pl.BlockSpec((ROW_BLK, N_INTENTS), lambda i: (i, 0)),
        out_shape=jax.ShapeDtypeStruct((n, N_INTENTS), jnp.float32),
    )(x, w)


def _normalize(x, eps=1e-12):
    n = jnp.linalg.norm(x, axis=1, keepdims=True)
    return x / jnp.maximum(n, eps)


def _spmm(values, h, t, mat):
    return jax.ops.segment_sum(values[:, None] * jnp.take(mat, t, axis=0), h,
                               num_segments=N_NODES)


def _adaptive_mask(head, tail, h):
    head = _normalize(head)
    tail = _normalize(tail)
    edge_alpha = (jnp.sum(head * tail, axis=1) + 1.0) / 2.0
    row_sum = jax.ops.segment_sum(edge_alpha, h, num_segments=N_NODES)
    d_inv = jnp.where(row_sum > 0, 1.0 / row_sum, 0.0)
    return jnp.take(d_inv, h) * edge_alpha


def kernel(user_emb, item_emb, user_intent, item_intent, G_values, all_h, all_t):
    all_emb = [jnp.concatenate([user_emb, item_emb], axis=0)]
    gnn_l, int_l, gaa_l, iaa_l = [], [], [], []
    for i in range(N_LAYERS):
        gnn = _spmm(G_values, all_h, all_t, all_emb[i])
        u_int = _intent(all_emb[i][:N_USERS], user_intent)
        i_int = _intent(all_emb[i][N_USERS:], item_intent)
        int_e = jnp.concatenate([u_int, i_int], axis=0)
        g_vals = _adaptive_mask(jnp.take(gnn, all_h, axis=0),
                                jnp.take(gnn, all_t, axis=0), all_h)
        i_vals = _adaptive_mask(jnp.take(int_e, all_h, axis=0),
                                jnp.take(int_e, all_t, axis=0), all_h)
        gaa = _spmm(g_vals, all_h, all_t, all_emb[i])
        iaa = _spmm(i_vals, all_h, all_t, all_emb[i])
        gnn_l.append(gnn)
        int_l.append(int_e)
        gaa_l.append(gaa)
        iaa_l.append(iaa)
        all_emb.append(gnn + int_e + gaa + iaa + all_emb[i])
    return jnp.stack([jnp.stack(gnn_l), jnp.stack(int_l),
                      jnp.stack(gaa_l), jnp.stack(iaa_l)])


# speedup vs baseline: 1.2012x; 1.1962x over previous
import functools

import jax
import jax.numpy as jnp
from jax import lax
from jax.experimental import pallas as pl
from jax.experimental.pallas import tpu as pltpu
from jax.experimental.pallas import tpu_sc as plsc

N_USERS = 5000
N_ITEMS = 5000
N_NODES = N_USERS + N_ITEMS
E_EDGES = 320000
D = 128
N_INTENTS = 128
N_LAYERS = 2

ROW_BLK = 1000

NC = 2
NS = 16
NW = NC * NS
CHUNK = 80
ZROWS = N_NODES // NS
ZBLK = 125


def _spmm_kernel(vals_hbm, h_hbm, t_hbm, mat_hbm, out_hbm,
                 tbuf, hbuf, vbuf, rows, stage, acc, sem):
    cid = lax.axis_index("c")
    sid = lax.axis_index("s")
    wid = sid * NC + cid
    per_w = E_EDGES // NW
    n_chunks = per_w // CHUNK

    def zrow(i, _):
        for k in range(D // 16):
            stage[i, pl.ds(k * 16, 16)] = jnp.zeros((16,), jnp.float32)
        return 0
    lax.fori_loop(0, ZBLK, zrow, 0)

    def zcopy(j, _):
        pltpu.sync_copy(stage, acc.at[pl.ds(sid * ZROWS + j * ZBLK, ZBLK)])
        return 0
    lax.fori_loop(0, ZROWS // ZBLK, zcopy, 0)
    plsc.subcore_barrier()

    base = wid * per_w

    def step(j, _):
        off = base + j * CHUNK
        pltpu.sync_copy(t_hbm.at[pl.ds(off, CHUNK)], tbuf)
        pltpu.sync_copy(h_hbm.at[pl.ds(off, CHUNK)], hbuf)
        pltpu.sync_copy(vals_hbm.at[pl.ds(off, CHUNK)], vbuf)
        pltpu.async_copy(mat_hbm.at[tbuf], rows, sem).wait()

        def sgrp(g, _):
            vseg = vbuf[pl.ds(g * 16, 16)]
            for l in range(16):
                v = vseg[l]
                i = g * 16 + l
                for k in range(D // 16):
                    sl = pl.ds(k * 16, 16)
                    rows[i, sl] = rows[i, sl] * v
            return 0
        lax.fori_loop(0, CHUNK // 16, sgrp, 0)
        pltpu.sync_copy(rows, acc.at[hbuf], add=True)
        return 0
    lax.fori_loop(0, n_chunks, step, 0)
    plsc.subcore_barrier()

    pltpu.sync_copy(acc.at[pl.ds(sid * ZROWS, ZROWS)], out_hbm.at[cid, sid])


def _spmm_sc(vals, h, t, mat):
    mesh = plsc.VectorSubcoreMesh(core_axis_name="c", subcore_axis_name="s",
                                  num_cores=NC, num_subcores=NS)
    f = pl.kernel(
        _spmm_kernel,
        out_type=jax.ShapeDtypeStruct((NC, NS, ZROWS, D), jnp.float32),
        mesh=mesh,
        scratch_types=[
            pltpu.VMEM((CHUNK,), jnp.int32),
            pltpu.VMEM((CHUNK,), jnp.int32),
            pltpu.VMEM((CHUNK,), jnp.float32),
            pltpu.VMEM((CHUNK, D), jnp.float32),
            pltpu.VMEM((ZBLK, D), jnp.float32),
            pltpu.VMEM_SHARED((N_NODES, D), jnp.float32),
            pltpu.SemaphoreType.DMA,
        ],
    )
    part = f(vals, h, t, mat)
    part = part.reshape(NC, N_NODES, D)
    return part[0] + part[1]


def _intent_body(x_ref, w_ref, o_ref):
    x = x_ref[...]
    w = w_ref[...]
    logits = jnp.dot(x, w, preferred_element_type=jnp.float32)
    m = jnp.max(logits, axis=1, keepdims=True)
    p = jnp.exp(logits - m)
    s = jnp.sum(p, axis=1, keepdims=True)
    p = p / s
    o_ref[...] = lax.dot_general(p, w, (((1,), (1,)), ((), ())),
                                 preferred_element_type=jnp.float32)


def _intent(x, w):
    n = x.shape[0]
    grid = n // ROW_BLK
    return pl.pallas_call(
        _intent_body,
        grid=(grid,),
        in_specs=[
            pl.BlockSpec((ROW_BLK, D), lambda i: (i, 0)),
            pl.BlockSpec((D, N_INTENTS), lambda i: (0, 0)),
        ],
        out_specs=pl.BlockSpec((ROW_BLK, N_INTENTS), lambda i: (i, 0)),
        out_shape=jax.ShapeDtypeStruct((n, N_INTENTS), jnp.float32),
    )(x, w)


def _normalize(x, eps=1e-12):
    n = jnp.linalg.norm(x, axis=1, keepdims=True)
    return x / jnp.maximum(n, eps)


def _spmm(values, h, t, mat):
    return _spmm_sc(values, h, t, mat)


def _adaptive_mask(head, tail, h):
    head = _normalize(head)
    tail = _normalize(tail)
    edge_alpha = (jnp.sum(head * tail, axis=1) + 1.0) / 2.0
    row_sum = jax.ops.segment_sum(edge_alpha, h, num_segments=N_NODES)
    d_inv = jnp.where(row_sum > 0, 1.0 / row_sum, 0.0)
    return jnp.take(d_inv, h) * edge_alpha


def kernel(user_emb, item_emb, user_intent, item_intent, G_values, all_h, all_t):
    all_emb = [jnp.concatenate([user_emb, item_emb], axis=0)]
    gnn_l, int_l, gaa_l, iaa_l = [], [], [], []
    for i in range(N_LAYERS):
        gnn = _spmm(G_values, all_h, all_t, all_emb[i])
        u_int = _intent(all_emb[i][:N_USERS], user_intent)
        i_int = _intent(all_emb[i][N_USERS:], item_intent)
        int_e = jnp.concatenate([u_int, i_int], axis=0)
        g_vals = _adaptive_mask(jnp.take(gnn, all_h, axis=0),
                                jnp.take(gnn, all_t, axis=0), all_h)
        i_vals = _adaptive_mask(jnp.take(int_e, all_h, axis=0),
                                jnp.take(int_e, all_t, axis=0), all_h)
        gaa = _spmm(g_vals, all_h, all_t, all_emb[i])
        iaa = _spmm(i_vals, all_h, all_t, all_emb[i])
        gnn_l.append(gnn)
        int_l.append(int_e)
        gaa_l.append(gaa)
        iaa_l.append(iaa)
        all_emb.append(gnn + int_e + gaa + iaa + all_emb[i])
    return jnp.stack([jnp.stack(gnn_l), jnp.stack(int_l),
                      jnp.stack(gaa_l), jnp.stack(iaa_l)])


# speedup vs baseline: 2.7334x; 2.2756x over previous
import functools

import jax
import jax.numpy as jnp
from jax import lax
from jax.experimental import pallas as pl
from jax.experimental.pallas import tpu as pltpu
from jax.experimental.pallas import tpu_sc as plsc

N_USERS = 5000
N_ITEMS = 5000
N_NODES = N_USERS + N_ITEMS
E_EDGES = 320000
D = 128
N_INTENTS = 128
N_LAYERS = 2

ROW_BLK = 1000

NC = 2
NS = 16
NW = NC * NS
CHUNK = 80
ZROWS = N_NODES // NS
ZBLK = 125


def _spmm_kernel(vals_hbm, h_hbm, t_hbm, mat_hbm, out_hbm,
                 tbuf, hbuf, vbuf, rows, stage, acc, sem):
    cid = lax.axis_index("c")
    sid = lax.axis_index("s")
    wid = sid * NC + cid
    per_w = E_EDGES // NW
    n_chunks = per_w // CHUNK

    def zrow(i, _):
        for k in range(D // 16):
            stage[i, pl.ds(k * 16, 16)] = jnp.zeros((16,), jnp.float32)
        return 0
    lax.fori_loop(0, ZBLK, zrow, 0)

    def zcopy(j, _):
        pltpu.sync_copy(stage, acc.at[pl.ds(sid * ZROWS + j * ZBLK, ZBLK)])
        return 0
    lax.fori_loop(0, ZROWS // ZBLK, zcopy, 0)
    plsc.subcore_barrier()

    base = wid * per_w

    def step(j, _):
        off = base + j * CHUNK
        pltpu.sync_copy(t_hbm.at[pl.ds(off, CHUNK)], tbuf)
        pltpu.sync_copy(h_hbm.at[pl.ds(off, CHUNK)], hbuf)
        pltpu.sync_copy(vals_hbm.at[pl.ds(off, CHUNK)], vbuf)
        pltpu.async_copy(mat_hbm.at[tbuf], rows, sem).wait()

        def sgrp(g, _):
            vseg = vbuf[pl.ds(g * 16, 16)]
            for l in range(16):
                v = vseg[l]
                i = g * 16 + l
                for k in range(D // 16):
                    sl = pl.ds(k * 16, 16)
                    rows[i, sl] = rows[i, sl] * v
            return 0
        lax.fori_loop(0, CHUNK // 16, sgrp, 0)
        pltpu.sync_copy(rows, acc.at[hbuf], add=True)
        return 0
    lax.fori_loop(0, n_chunks, step, 0)
    plsc.subcore_barrier()

    pltpu.sync_copy(acc.at[pl.ds(sid * ZROWS, ZROWS)], out_hbm.at[cid, sid])


def _spmm_sc(vals, h, t, mat):
    mesh = plsc.VectorSubcoreMesh(core_axis_name="c", subcore_axis_name="s",
                                  num_cores=NC, num_subcores=NS)
    f = pl.kernel(
        _spmm_kernel,
        out_type=jax.ShapeDtypeStruct((NC, NS, ZROWS, D), jnp.float32),
        mesh=mesh,
        scratch_types=[
            pltpu.VMEM((CHUNK,), jnp.int32),
            pltpu.VMEM((CHUNK,), jnp.int32),
            pltpu.VMEM((CHUNK,), jnp.float32),
            pltpu.VMEM((CHUNK, D), jnp.float32),
            pltpu.VMEM((ZBLK, D), jnp.float32),
            pltpu.VMEM_SHARED((N_NODES, D), jnp.float32),
            pltpu.SemaphoreType.DMA,
        ],
    )
    part = f(vals, h, t, mat)
    part = part.reshape(NC, N_NODES, D)
    return part[0] + part[1]


NPAD = 10240
PERW = E_EDGES // NW
NCHUNK = PERW // CHUNK


def _dyn16(v, idx):
    dnums = lax.GatherDimensionNumbers(
        offset_dims=(), collapsed_slice_dims=(0,), start_index_map=(0,))
    return lax.gather(v, idx[:, None], dnums, (1,),
                      mode=lax.GatherScatterMode.PROMISE_IN_BOUNDS)


def _mask_sc_kernel(ng_hbm, ni_hbm, h_hbm, t_hbm, ag_hbm, ai_hbm,
                    rsg_hbm, rsi_hbm,
                    hbuf, tbuf, ngh, ngt, nih, nit, agbuf, aibuf,
                    rsg1, rsi1, sem):
    cid = lax.axis_index("c")
    sid = lax.axis_index("s")
    wid = sid * NC + cid
    lanes = lax.iota(jnp.int32, 16)

    def zrow(i, _):
        sl = pl.ds(i * 16, 16)
        rsg1[sl] = jnp.zeros((16,), jnp.float32)
        rsi1[sl] = jnp.zeros((16,), jnp.float32)
        return 0
    lax.fori_loop(0, NPAD // 16, zrow, 0)

    base = wid * PERW

    def step(j, _):
        off = base + j * CHUNK
        pltpu.sync_copy(h_hbm.at[pl.ds(off, CHUNK)], hbuf)
        pltpu.sync_copy(t_hbm.at[pl.ds(off, CHUNK)], tbuf)
        d1 = pltpu.async_copy(ng_hbm.at[hbuf], ngh, sem)
        d2 = pltpu.async_copy(ng_hbm.at[tbuf], ngt, sem)
        d3 = pltpu.async_copy(ni_hbm.at[hbuf], nih, sem)
        d4 = pltpu.async_copy(ni_hbm.at[tbuf], nit, sem)
        d1.wait()
        d2.wait()
        d3.wait()
        d4.wait()

        def grp(g, _):
            row0 = g * 16
            ridx = row0 + lanes

            def col(dd, carry):
                ag_, ai_ = carry
                cidx = jnp.full((16,), dd, jnp.int32)
                ag_ = ag_ + (plsc.load_gather(ngh, [ridx, cidx]) *
                             plsc.load_gather(ngt, [ridx, cidx]))
                ai_ = ai_ + (plsc.load_gather(nih, [ridx, cidx]) *
                             plsc.load_gather(nit, [ridx, cidx]))
                return ag_, ai_

            accg, acci = lax.fori_loop(
                0, D, col,
                (jnp.zeros((16,), jnp.float32), jnp.zeros((16,), jnp.float32)),
                unroll=8)
            alpha_g = (accg + 1.0) * 0.5
            alpha_i = (acci + 1.0) * 0.5
            agbuf[pl.ds(row0, 16)] = alpha_g
            aibuf[pl.ds(row0, 16)] = alpha_i

            hv = hbuf[pl.ds(row0, 16)]
            ks, perm = plsc.sort_key_val(hv, lanes)
            ag_s = _dyn16(alpha_g, perm)
            ai_s = _dyn16(alpha_i, perm)
            cg = plsc.cumsum(ag_s)
            ci = plsc.cumsum(ai_s)
            eg = cg - ag_s
            ei = ci - ai_s
            kp = _dyn16(ks, jnp.maximum(lanes - 1, 0))
            m_first = jnp.logical_or(jnp.not_equal(ks, kp), lanes == 0)
            zero = jnp.zeros((16,), jnp.float32)
            bg = plsc.cummax(jnp.where(m_first, eg, zero))
            bi = plsc.cummax(jnp.where(m_first, ei, zero))
            kn = _dyn16(ks, jnp.minimum(lanes + 1, 15))
            m_last = jnp.logical_or(jnp.not_equal(ks, kn), lanes == 15)
            plsc.addupdate_scatter(rsg1, [ks], cg - bg, mask=m_last)
            plsc.addupdate_scatter(rsi1, [ks], ci - bi, mask=m_last)
            return 0

        lax.fori_loop(0, CHUNK // 16, grp, 0)
        pltpu.sync_copy(agbuf, ag_hbm.at[pl.ds(off, CHUNK)])
        pltpu.sync_copy(aibuf, ai_hbm.at[pl.ds(off, CHUNK)])
        return 0

    lax.fori_loop(0, NCHUNK, step, 0)
    pltpu.sync_copy(rsg1, rsg_hbm.at[cid, sid, 0])
    pltpu.sync_copy(rsi1, rsi_hbm.at[cid, sid, 0])


def _mask_sc(ng, ni, h, t):
    mesh = plsc.VectorSubcoreMesh(core_axis_name="c", subcore_axis_name="s",
                                  num_cores=NC, num_subcores=NS)
    f = pl.kernel(
        _mask_sc_kernel,
        out_type=(
            jax.ShapeDtypeStruct((E_EDGES,), jnp.float32),
            jax.ShapeDtypeStruct((E_EDGES,), jnp.float32),
            jax.ShapeDtypeStruct((NC, NS, 1, NPAD), jnp.float32),
            jax.ShapeDtypeStruct((NC, NS, 1, NPAD), jnp.float32),
        ),
        mesh=mesh,
        scratch_types=[
            pltpu.VMEM((CHUNK,), jnp.int32),
            pltpu.VMEM((CHUNK,), jnp.int32),
            pltpu.VMEM((CHUNK, D), jnp.float32),
            pltpu.VMEM((CHUNK, D), jnp.float32),
            pltpu.VMEM((CHUNK, D), jnp.float32),
            pltpu.VMEM((CHUNK, D), jnp.float32),
            pltpu.VMEM((CHUNK,), jnp.float32),
            pltpu.VMEM((CHUNK,), jnp.float32),
            pltpu.VMEM((NPAD,), jnp.float32),
            pltpu.VMEM((NPAD,), jnp.float32),
            pltpu.SemaphoreType.DMA,
        ],
        compiler_params=pltpu.CompilerParams(needs_layout_passes=False),
    )
    ag, ai, rsg, rsi = f(ng, ni, h, t)
    rs_g = jnp.sum(rsg.reshape(NW, NPAD), axis=0)[:N_NODES]
    rs_i = jnp.sum(rsi.reshape(NW, NPAD), axis=0)[:N_NODES]
    return ag, ai, rs_g, rs_i


def _intent_body(x_ref, w_ref, o_ref):
    x = x_ref[...]
    w = w_ref[...]
    logits = jnp.dot(x, w, preferred_element_type=jnp.float32)
    m = jnp.max(logits, axis=1, keepdims=True)
    p = jnp.exp(logits - m)
    s = jnp.sum(p, axis=1, keepdims=True)
    p = p / s
    o_ref[...] = lax.dot_general(p, w, (((1,), (1,)), ((), ())),
                                 preferred_element_type=jnp.float32)


def _intent(x, w):
    n = x.shape[0]
    grid = n // ROW_BLK
    return pl.pallas_call(
        _intent_body,
        grid=(grid,),
        in_specs=[
            pl.BlockSpec((ROW_BLK, D), lambda i: (i, 0)),
            pl.BlockSpec((D, N_INTENTS), lambda i: (0, 0)),
        ],
        out_specs=pl.BlockSpec((ROW_BLK, N_INTENTS), lambda i: (i, 0)),
        out_shape=jax.ShapeDtypeStruct((n, N_INTENTS), jnp.float32),
    )(x, w)


def _tc_b_body(gnn_ref, emb_ref, wu_ref, wi_ref, ng_ref, int_ref, ni_ref):
    pid = pl.program_id(0)
    g = gnn_ref[...]
    n2 = jnp.sum(g * g, axis=1, keepdims=True)
    ng_ref[...] = g / jnp.maximum(jnp.sqrt(n2), 1e-12)
    x = emb_ref[...]
    w = jnp.where(pid < (N_USERS // ROW_BLK), wu_ref[...], wi_ref[...])
    logits = jnp.dot(x, w, preferred_element_type=jnp.float32)
    m = jnp.max(logits, axis=1, keepdims=True)
    p = jnp.exp(logits - m)
    p = p / jnp.sum(p, axis=1, keepdims=True)
    it = lax.dot_general(p, w, (((1,), (1,)), ((), ())),
                         preferred_element_type=jnp.float32)
    int_ref[...] = it
    n2i = jnp.sum(it * it, axis=1, keepdims=True)
    ni_ref[...] = it / jnp.maximum(jnp.sqrt(n2i), 1e-12)


def _tc_b(gnn, emb, wu, wi):
    blk = pl.BlockSpec((ROW_BLK, D), lambda i: (i, 0))
    wspec = pl.BlockSpec((D, N_INTENTS), lambda i: (0, 0))
    return pl.pallas_call(
        _tc_b_body,
        grid=(N_NODES // ROW_BLK,),
        in_specs=[blk, blk, wspec, wspec],
        out_specs=[blk, blk, blk],
        out_shape=[jax.ShapeDtypeStruct((N_NODES, D), jnp.float32)] * 3,
    )(gnn, emb, wu, wi)


def _tc_e_body(gp_ref, ip_ref, dg_ref, di_ref, gnn_ref, int_ref, emb_ref,
               gaa_ref, iaa_ref, nxt_ref):
    gaa = dg_ref[...] * gp_ref[...]
    iaa = di_ref[...] * ip_ref[...]
    gaa_ref[...] = gaa
    iaa_ref[...] = iaa
    nxt_ref[...] = gnn_ref[...] + int_ref[...] + gaa + iaa + emb_ref[...]


def _tc_e(gaap, iaap, dg, di, gnn, int_e, emb):
    blk = pl.BlockSpec((ROW_BLK, D), lambda i: (i, 0))
    dspec = pl.BlockSpec((ROW_BLK, 1), lambda i: (i, 0))
    return pl.pallas_call(
        _tc_e_body,
        grid=(N_NODES // ROW_BLK,),
        in_specs=[blk, blk, dspec, dspec, blk, blk, blk],
        out_specs=[blk, blk, blk],
        out_shape=[jax.ShapeDtypeStruct((N_NODES, D), jnp.float32)] * 3,
    )(gaap, iaap, dg[:, None], di[:, None], gnn, int_e, emb)


def kernel(user_emb, item_emb, user_intent, item_intent, G_values, all_h, all_t):
    emb = jnp.concatenate([user_emb, item_emb], axis=0)
    gnn_l, int_l, gaa_l, iaa_l = [], [], [], []
    for i in range(N_LAYERS):
        gnn = _spmm_sc(G_values, all_h, all_t, emb)
        ng, int_e, ni = _tc_b(gnn, emb, user_intent, item_intent)
        ag, ai, rsg, rsi = _mask_sc(ng, ni, all_h, all_t)
        gaap = _spmm_sc(ag, all_h, all_t, emb)
        iaap = _spmm_sc(ai, all_h, all_t, emb)
        dg = jnp.where(rsg > 0, 1.0 / rsg, 0.0)
        di = jnp.where(rsi > 0, 1.0 / rsi, 0.0)
        gaa, iaa, emb_next = _tc_e(gaap, iaap, dg, di, gnn, int_e, emb)
        gnn_l.append(gnn)
        int_l.append(int_e)
        gaa_l.append(gaa)
        iaa_l.append(iaa)
        emb = emb_next
    return jnp.stack([jnp.stack(gnn_l), jnp.stack(int_l),
                      jnp.stack(gaa_l), jnp.stack(iaa_l)])


# speedup vs baseline: 6.3010x; 2.3052x over previous
import functools

import jax
import jax.numpy as jnp
from jax import lax
from jax.experimental import pallas as pl
from jax.experimental.pallas import tpu as pltpu
from jax.experimental.pallas import tpu_sc as plsc

N_USERS = 5000
N_ITEMS = 5000
N_NODES = N_USERS + N_ITEMS
E_EDGES = 320000
D = 128
N_INTENTS = 128
N_LAYERS = 2

ROW_BLK = 1000

NC = 2
NS = 16
NW = NC * NS
CHUNK = 80
ZROWS = N_NODES // NS
ZBLK = 125


def _spmm_kernel(vals_hbm, h_hbm, t_hbm, mat_hbm, out_hbm,
                 tbuf, hbuf, vbuf, rows, stage, acc, sem):
    cid = lax.axis_index("c")
    sid = lax.axis_index("s")
    wid = sid * NC + cid
    per_w = E_EDGES // NW
    n_chunks = per_w // CHUNK

    def zrow(i, _):
        for k in range(D // 16):
            stage[i, pl.ds(k * 16, 16)] = jnp.zeros((16,), jnp.float32)
        return 0
    lax.fori_loop(0, ZBLK, zrow, 0)

    def zcopy(j, _):
        pltpu.sync_copy(stage, acc.at[pl.ds(sid * ZROWS + j * ZBLK, ZBLK)])
        return 0
    lax.fori_loop(0, ZROWS // ZBLK, zcopy, 0)
    plsc.subcore_barrier()

    base = wid * per_w

    def step(j, _):
        off = base + j * CHUNK
        pltpu.sync_copy(t_hbm.at[pl.ds(off, CHUNK)], tbuf)
        pltpu.sync_copy(h_hbm.at[pl.ds(off, CHUNK)], hbuf)
        pltpu.sync_copy(vals_hbm.at[pl.ds(off, CHUNK)], vbuf)
        pltpu.async_copy(mat_hbm.at[tbuf], rows, sem).wait()

        def sgrp(g, _):
            vseg = vbuf[pl.ds(g * 16, 16)]
            for l in range(16):
                v = vseg[l]
                i = g * 16 + l
                for k in range(D // 16):
                    sl = pl.ds(k * 16, 16)
                    rows[i, sl] = rows[i, sl] * v
            return 0
        lax.fori_loop(0, CHUNK // 16, sgrp, 0)
        pltpu.sync_copy(rows, acc.at[hbuf], add=True)
        return 0
    lax.fori_loop(0, n_chunks, step, 0)
    plsc.subcore_barrier()

    pltpu.sync_copy(acc.at[pl.ds(sid * ZROWS, ZROWS)], out_hbm.at[cid, sid])


def _spmm_sc(vals, h, t, mat):
    mesh = plsc.VectorSubcoreMesh(core_axis_name="c", subcore_axis_name="s",
                                  num_cores=NC, num_subcores=NS)
    f = pl.kernel(
        _spmm_kernel,
        out_type=jax.ShapeDtypeStruct((NC, NS, ZROWS, D), jnp.float32),
        mesh=mesh,
        scratch_types=[
            pltpu.VMEM((CHUNK,), jnp.int32),
            pltpu.VMEM((CHUNK,), jnp.int32),
            pltpu.VMEM((CHUNK,), jnp.float32),
            pltpu.VMEM((CHUNK, D), jnp.float32),
            pltpu.VMEM((ZBLK, D), jnp.float32),
            pltpu.VMEM_SHARED((N_NODES, D), jnp.float32),
            pltpu.SemaphoreType.DMA,
        ],
    )
    part = f(vals, h, t, mat)
    part = part.reshape(NC, N_NODES, D)
    return part[0] + part[1]


NPAD = 10240
PERW = E_EDGES // NW
NCHUNK = PERW // CHUNK


def _dyn16(v, idx):
    dnums = lax.GatherDimensionNumbers(
        offset_dims=(), collapsed_slice_dims=(0,), start_index_map=(0,))
    return lax.gather(v, idx[:, None], dnums, (1,),
                      mode=lax.GatherScatterMode.PROMISE_IN_BOUNDS)


def _mask_sc_kernel(m_hbm, h_hbm, t_hbm, ag_hbm, ai_hbm,
                    rsg_hbm, rsi_hbm,
                    hbuf, tbuf, mh, mt, agbuf, aibuf,
                    rsg1, rsi1, sem):
    cid = lax.axis_index("c")
    sid = lax.axis_index("s")
    wid = sid * NC + cid
    lanes = lax.iota(jnp.int32, 16)
    himask = jnp.full((16,), -65536, jnp.int32)
    perms = [lanes ^ sh for sh in (8, 4, 2, 1)]
    cols = [k * 16 + lanes for k in range(D // 16)]

    def hsum(v):
        for p in perms:
            v = v + _dyn16(v, p)
        return v

    def zrow(i, _):
        sl = pl.ds(i * 16, 16)
        rsg1[sl] = jnp.zeros((16,), jnp.float32)
        rsi1[sl] = jnp.zeros((16,), jnp.float32)
        return 0
    lax.fori_loop(0, NPAD // 16, zrow, 0)

    base = wid * PERW

    def step(j, _):
        off = base + j * CHUNK
        pltpu.sync_copy(h_hbm.at[pl.ds(off, CHUNK)], hbuf)
        pltpu.sync_copy(t_hbm.at[pl.ds(off, CHUNK)], tbuf)
        d1 = pltpu.async_copy(m_hbm.at[hbuf], mh, sem)
        d2 = pltpu.async_copy(m_hbm.at[tbuf], mt, sem)
        d1.wait()
        d2.wait()

        def grp(g, _):
            row0 = g * 16
            alpha_g = jnp.zeros((16,), jnp.float32)
            alpha_i = jnp.zeros((16,), jnp.float32)
            for l in range(16):
                row = row0 + l
                rsplat = jnp.full((16,), row, jnp.int32)
                accg = jnp.zeros((16,), jnp.float32)
                acci = jnp.zeros((16,), jnp.float32)
                for k in range(D // 16):
                    bh = plsc.bitcast(
                        plsc.load_gather(mh, [rsplat, cols[k]]), jnp.int32)
                    bt = plsc.bitcast(
                        plsc.load_gather(mt, [rsplat, cols[k]]), jnp.int32)
                    hg = plsc.bitcast(bh & himask, jnp.float32)
                    tg = plsc.bitcast(bt & himask, jnp.float32)
                    hi = plsc.bitcast(bh << 16, jnp.float32)
                    ti = plsc.bitcast(bt << 16, jnp.float32)
                    accg = accg + hg * tg
                    acci = acci + hi * ti
                lmask = lanes == l
                alpha_g = jnp.where(lmask, (hsum(accg) + 1.0) * 0.5, alpha_g)
                alpha_i = jnp.where(lmask, (hsum(acci) + 1.0) * 0.5, alpha_i)
            agbuf[pl.ds(row0, 16)] = alpha_g
            aibuf[pl.ds(row0, 16)] = alpha_i

            hv = hbuf[pl.ds(row0, 16)]
            ks, perm = plsc.sort_key_val(hv, lanes)
            ag_s = _dyn16(alpha_g, perm)
            ai_s = _dyn16(alpha_i, perm)
            cg = plsc.cumsum(ag_s)
            ci = plsc.cumsum(ai_s)
            eg = cg - ag_s
            ei = ci - ai_s
            kp = _dyn16(ks, jnp.maximum(lanes - 1, 0))
            m_first = jnp.logical_or(jnp.not_equal(ks, kp), lanes == 0)
            zero = jnp.zeros((16,), jnp.float32)
            bg = plsc.cummax(jnp.where(m_first, eg, zero))
            bi = plsc.cummax(jnp.where(m_first, ei, zero))
            kn = _dyn16(ks, jnp.minimum(lanes + 1, 15))
            m_last = jnp.logical_or(jnp.not_equal(ks, kn), lanes == 15)
            plsc.addupdate_scatter(rsg1, [ks], cg - bg, mask=m_last)
            plsc.addupdate_scatter(rsi1, [ks], ci - bi, mask=m_last)
            return 0

        lax.fori_loop(0, CHUNK // 16, grp, 0)
        pltpu.sync_copy(agbuf, ag_hbm.at[pl.ds(off, CHUNK)])
        pltpu.sync_copy(aibuf, ai_hbm.at[pl.ds(off, CHUNK)])
        return 0

    lax.fori_loop(0, NCHUNK, step, 0)
    pltpu.sync_copy(rsg1, rsg_hbm.at[cid, sid, 0])
    pltpu.sync_copy(rsi1, rsi_hbm.at[cid, sid, 0])


def _mask_sc(m_packed, h, t):
    mesh = plsc.VectorSubcoreMesh(core_axis_name="c", subcore_axis_name="s",
                                  num_cores=NC, num_subcores=NS)
    f = pl.kernel(
        _mask_sc_kernel,
        out_type=(
            jax.ShapeDtypeStruct((E_EDGES,), jnp.float32),
            jax.ShapeDtypeStruct((E_EDGES,), jnp.float32),
            jax.ShapeDtypeStruct((NC, NS, 1, NPAD), jnp.float32),
            jax.ShapeDtypeStruct((NC, NS, 1, NPAD), jnp.float32),
        ),
        mesh=mesh,
        scratch_types=[
            pltpu.VMEM((CHUNK,), jnp.int32),
            pltpu.VMEM((CHUNK,), jnp.int32),
            pltpu.VMEM((CHUNK, D), jnp.float32),
            pltpu.VMEM((CHUNK, D), jnp.float32),
            pltpu.VMEM((CHUNK,), jnp.float32),
            pltpu.VMEM((CHUNK,), jnp.float32),
            pltpu.VMEM((NPAD,), jnp.float32),
            pltpu.VMEM((NPAD,), jnp.float32),
            pltpu.SemaphoreType.DMA,
        ],
        compiler_params=pltpu.CompilerParams(needs_layout_passes=False),
    )
    ag, ai, rsg, rsi = f(m_packed, h, t)
    rs_g = jnp.sum(rsg.reshape(NW, NPAD), axis=0)[:N_NODES]
    rs_i = jnp.sum(rsi.reshape(NW, NPAD), axis=0)[:N_NODES]
    return ag, ai, rs_g, rs_i


def _intent_body(x_ref, w_ref, o_ref):
    x = x_ref[...]
    w = w_ref[...]
    logits = jnp.dot(x, w, preferred_element_type=jnp.float32)
    m = jnp.max(logits, axis=1, keepdims=True)
    p = jnp.exp(logits - m)
    s = jnp.sum(p, axis=1, keepdims=True)
    p = p / s
    o_ref[...] = lax.dot_general(p, w, (((1,), (1,)), ((), ())),
                                 preferred_element_type=jnp.float32)


def _intent(x, w):
    n = x.shape[0]
    grid = n // ROW_BLK
    return pl.pallas_call(
        _intent_body,
        grid=(grid,),
        in_specs=[
            pl.BlockSpec((ROW_BLK, D), lambda i: (i, 0)),
            pl.BlockSpec((D, N_INTENTS), lambda i: (0, 0)),
        ],
        out_specs=pl.BlockSpec((ROW_BLK, N_INTENTS), lambda i: (i, 0)),
        out_shape=jax.ShapeDtypeStruct((n, N_INTENTS), jnp.float32),
    )(x, w)


def _tc_b_body(gnn_ref, emb_ref, wu_ref, wi_ref, m_ref, int_ref):
    pid = pl.program_id(0)
    g = gnn_ref[...]
    n2 = jnp.sum(g * g, axis=1, keepdims=True)
    ng = g / jnp.maximum(jnp.sqrt(n2), 1e-12)
    x = emb_ref[...]
    w = jnp.where(pid < (N_USERS // ROW_BLK), wu_ref[...], wi_ref[...])
    logits = jnp.dot(x, w, preferred_element_type=jnp.float32)
    mx = jnp.max(logits, axis=1, keepdims=True)
    p = jnp.exp(logits - mx)
    p = p / jnp.sum(p, axis=1, keepdims=True)
    it = lax.dot_general(p, w, (((1,), (1,)), ((), ())),
                         preferred_element_type=jnp.float32)
    int_ref[...] = it
    n2i = jnp.sum(it * it, axis=1, keepdims=True)
    ni = it / jnp.maximum(jnp.sqrt(n2i), 1e-12)
    u_ng = lax.bitcast_convert_type(ng.astype(jnp.bfloat16),
                                    jnp.uint16).astype(jnp.uint32)
    u_ni = lax.bitcast_convert_type(ni.astype(jnp.bfloat16),
                                    jnp.uint16).astype(jnp.uint32)
    m_ref[...] = lax.bitcast_convert_type((u_ng << 16) | u_ni, jnp.float32)


def _tc_b(gnn, emb, wu, wi):
    blk = pl.BlockSpec((ROW_BLK, D), lambda i: (i, 0))
    wspec = pl.BlockSpec((D, N_INTENTS), lambda i: (0, 0))
    return pl.pallas_call(
        _tc_b_body,
        grid=(N_NODES // ROW_BLK,),
        in_specs=[blk, blk, wspec, wspec],
        out_specs=[blk, blk],
        out_shape=[jax.ShapeDtypeStruct((N_NODES, D), jnp.float32)] * 2,
    )(gnn, emb, wu, wi)


def _tc_e_body(gp_ref, ip_ref, dg_ref, di_ref, gnn_ref, int_ref, emb_ref,
               gaa_ref, iaa_ref, nxt_ref):
    gaa = dg_ref[...] * gp_ref[...]
    iaa = di_ref[...] * ip_ref[...]
    gaa_ref[...] = gaa
    iaa_ref[...] = iaa
    nxt_ref[...] = gnn_ref[...] + int_ref[...] + gaa + iaa + emb_ref[...]


def _tc_e(gaap, iaap, dg, di, gnn, int_e, emb):
    blk = pl.BlockSpec((ROW_BLK, D), lambda i: (i, 0))
    dspec = pl.BlockSpec((ROW_BLK, 1), lambda i: (i, 0))
    return pl.pallas_call(
        _tc_e_body,
        grid=(N_NODES // ROW_BLK,),
        in_specs=[blk, blk, dspec, dspec, blk, blk, blk],
        out_specs=[blk, blk, blk],
        out_shape=[jax.ShapeDtypeStruct((N_NODES, D), jnp.float32)] * 3,
    )(gaap, iaap, dg[:, None], di[:, None], gnn, int_e, emb)


def kernel(user_emb, item_emb, user_intent, item_intent, G_values, all_h, all_t):
    emb = jnp.concatenate([user_emb, item_emb], axis=0)
    gnn_l, int_l, gaa_l, iaa_l = [], [], [], []
    for i in range(N_LAYERS):
        gnn = _spmm_sc(G_values, all_h, all_t, emb)
        m_packed, int_e = _tc_b(gnn, emb, user_intent, item_intent)
        ag, ai, rsg, rsi = _mask_sc(m_packed, all_h, all_t)
        gaap = _spmm_sc(ag, all_h, all_t, emb)
        iaap = _spmm_sc(ai, all_h, all_t, emb)
        dg = jnp.where(rsg > 0, 1.0 / rsg, 0.0)
        di = jnp.where(rsi > 0, 1.0 / rsi, 0.0)
        gaa, iaa, emb_next = _tc_e(gaap, iaap, dg, di, gnn, int_e, emb)
        gnn_l.append(gnn)
        int_l.append(int_e)
        gaa_l.append(gaa)
        iaa_l.append(iaa)
        emb = emb_next
    return jnp.stack([jnp.stack(gnn_l), jnp.stack(int_l),
                      jnp.stack(gaa_l), jnp.stack(iaa_l)])


# speedup vs baseline: 7.9567x; 1.2628x over previous
import functools

import jax
import jax.numpy as jnp
from jax import lax
from jax.experimental import pallas as pl
from jax.experimental.pallas import tpu as pltpu
from jax.experimental.pallas import tpu_sc as plsc

N_USERS = 5000
N_ITEMS = 5000
N_NODES = N_USERS + N_ITEMS
E_EDGES = 320000
D = 128
N_INTENTS = 128
N_LAYERS = 2

ROW_BLK = 1000

NC = 2
NS = 16
NW = NC * NS
CHUNK = 80
ZROWS = N_NODES // NS
ZBLK = 125


def _spmm_kernel(vals_hbm, h_hbm, t_hbm, mat_hbm, out_hbm,
                 tb0, hb0, vb0, rw0, tb1, hb1, vb1, rw1, zbuf, acc, s0, s1):
    cid = lax.axis_index("c")
    sid = lax.axis_index("s")
    wid = sid * NC + cid
    tb, hb, vb, rw, sems = [tb0, tb1], [hb0, hb1], [vb0, vb1], [rw0, rw1], [s0, s1]

    def zrow(i, _):
        for k in range(D // 16):
            zbuf[i, pl.ds(k * 16, 16)] = jnp.zeros((16,), jnp.float32)
        return 0
    lax.fori_loop(0, ZBLK, zrow, 0)

    def zcopy(j, _):
        pltpu.sync_copy(zbuf, acc.at[pl.ds(sid * ZROWS + j * ZBLK, ZBLK)])
        return 0
    lax.fori_loop(0, ZROWS // ZBLK, zcopy, 0)
    plsc.subcore_barrier()

    base = wid * PERW

    def load_chunk(b, off):
        pltpu.sync_copy(t_hbm.at[pl.ds(off, CHUNK)], tb[b])
        pltpu.sync_copy(h_hbm.at[pl.ds(off, CHUNK)], hb[b])
        pltpu.sync_copy(vals_hbm.at[pl.ds(off, CHUNK)], vb[b])
        pltpu.async_copy(mat_hbm.at[tb[b]], rw[b], sems[b])

    def process(b):
        pltpu.make_async_copy(mat_hbm.at[tb[b]], rw[b], sems[b]).wait()
        rows = rw[b]

        def sgrp(g, _):
            vseg = vb[b][pl.ds(g * 16, 16)]
            for l in range(16):
                v = vseg[l]
                i = g * 16 + l
                for k in range(D // 16):
                    sl = pl.ds(k * 16, 16)
                    rows[i, sl] = rows[i, sl] * v
            return 0
        lax.fori_loop(0, CHUNK // 16, sgrp, 0)
        pltpu.sync_copy(rows, acc.at[hb[b]], add=True)

    load_chunk(0, base)

    def pair(p, _):
        c0 = 2 * p
        load_chunk(1, base + (c0 + 1) * CHUNK)
        process(0)

        @pl.when(c0 + 2 < NCHUNK)
        def _():
            load_chunk(0, base + (c0 + 2) * CHUNK)
        process(1)
        return 0
    lax.fori_loop(0, NCHUNK // 2, pair, 0)
    if NCHUNK % 2 == 1:
        process(0)
    plsc.subcore_barrier()

    pltpu.sync_copy(acc.at[pl.ds(sid * ZROWS, ZROWS)], out_hbm.at[cid, sid])


def _spmm_sc(vals, h, t, mat):
    mesh = plsc.VectorSubcoreMesh(core_axis_name="c", subcore_axis_name="s",
                                  num_cores=NC, num_subcores=NS)
    f = pl.kernel(
        _spmm_kernel,
        out_type=jax.ShapeDtypeStruct((NC, NS, ZROWS, D), jnp.float32),
        mesh=mesh,
        scratch_types=[
            pltpu.VMEM((CHUNK,), jnp.int32),
            pltpu.VMEM((CHUNK,), jnp.int32),
            pltpu.VMEM((CHUNK,), jnp.float32),
            pltpu.VMEM((CHUNK, D), jnp.float32),
            pltpu.VMEM((CHUNK,), jnp.int32),
            pltpu.VMEM((CHUNK,), jnp.int32),
            pltpu.VMEM((CHUNK,), jnp.float32),
            pltpu.VMEM((CHUNK, D), jnp.float32),
            pltpu.VMEM((ZBLK, D), jnp.float32),
            pltpu.VMEM_SHARED((N_NODES, D), jnp.float32),
            pltpu.SemaphoreType.DMA,
            pltpu.SemaphoreType.DMA,
        ],
    )
    part = f(vals, h, t, mat)
    part = part.reshape(NC, N_NODES, D)
    return part[0] + part[1]


NPAD = 10240
PERW = E_EDGES // NW
NCHUNK = PERW // CHUNK


def _dyn16(v, idx):
    dnums = lax.GatherDimensionNumbers(
        offset_dims=(), collapsed_slice_dims=(0,), start_index_map=(0,))
    return lax.gather(v, idx[:, None], dnums, (1,),
                      mode=lax.GatherScatterMode.PROMISE_IN_BOUNDS)


def _mask_sc_kernel(m_hbm, h_hbm, t_hbm, ag_hbm, ai_hbm,
                    rsg_hbm, rsi_hbm,
                    hbuf, tbuf, mh, mt, agbuf, aibuf,
                    rsg1, rsi1, sem):
    cid = lax.axis_index("c")
    sid = lax.axis_index("s")
    wid = sid * NC + cid
    lanes = lax.iota(jnp.int32, 16)
    himask = jnp.full((16,), -65536, jnp.int32)
    perms = [lanes ^ sh for sh in (8, 4, 2, 1)]
    cols = [k * 16 + lanes for k in range(D // 16)]

    def hsum(v):
        for p in perms:
            v = v + _dyn16(v, p)
        return v

    def zrow(i, _):
        sl = pl.ds(i * 16, 16)
        rsg1[sl] = jnp.zeros((16,), jnp.float32)
        rsi1[sl] = jnp.zeros((16,), jnp.float32)
        return 0
    lax.fori_loop(0, NPAD // 16, zrow, 0)

    base = wid * PERW

    def step(j, _):
        off = base + j * CHUNK
        pltpu.sync_copy(h_hbm.at[pl.ds(off, CHUNK)], hbuf)
        pltpu.sync_copy(t_hbm.at[pl.ds(off, CHUNK)], tbuf)
        d1 = pltpu.async_copy(m_hbm.at[hbuf], mh, sem)
        d2 = pltpu.async_copy(m_hbm.at[tbuf], mt, sem)
        d1.wait()
        d2.wait()

        def grp(g, _):
            row0 = g * 16
            alpha_g = jnp.zeros((16,), jnp.float32)
            alpha_i = jnp.zeros((16,), jnp.float32)
            for l in range(16):
                row = row0 + l
                rsplat = jnp.full((16,), row, jnp.int32)
                accg = jnp.zeros((16,), jnp.float32)
                acci = jnp.zeros((16,), jnp.float32)
                for k in range(D // 16):
                    bh = plsc.bitcast(
                        plsc.load_gather(mh, [rsplat, cols[k]]), jnp.int32)
                    bt = plsc.bitcast(
                        plsc.load_gather(mt, [rsplat, cols[k]]), jnp.int32)
                    hg = plsc.bitcast(bh & himask, jnp.float32)
                    tg = plsc.bitcast(bt & himask, jnp.float32)
                    hi = plsc.bitcast(bh << 16, jnp.float32)
                    ti = plsc.bitcast(bt << 16, jnp.float32)
                    accg = accg + hg * tg
                    acci = acci + hi * ti
                lmask = lanes == l
                alpha_g = jnp.where(lmask, (hsum(accg) + 1.0) * 0.5, alpha_g)
                alpha_i = jnp.where(lmask, (hsum(acci) + 1.0) * 0.5, alpha_i)
            agbuf[pl.ds(row0, 16)] = alpha_g
            aibuf[pl.ds(row0, 16)] = alpha_i

            hv = hbuf[pl.ds(row0, 16)]
            ks, perm = plsc.sort_key_val(hv, lanes)
            ag_s = _dyn16(alpha_g, perm)
            ai_s = _dyn16(alpha_i, perm)
            cg = plsc.cumsum(ag_s)
            ci = plsc.cumsum(ai_s)
            eg = cg - ag_s
            ei = ci - ai_s
            kp = _dyn16(ks, jnp.maximum(lanes - 1, 0))
            m_first = jnp.logical_or(jnp.not_equal(ks, kp), lanes == 0)
            zero = jnp.zeros((16,), jnp.float32)
            bg = plsc.cummax(jnp.where(m_first, eg, zero))
            bi = plsc.cummax(jnp.where(m_first, ei, zero))
            kn = _dyn16(ks, jnp.minimum(lanes + 1, 15))
            m_last = jnp.logical_or(jnp.not_equal(ks, kn), lanes == 15)
            plsc.addupdate_scatter(rsg1, [ks], cg - bg, mask=m_last)
            plsc.addupdate_scatter(rsi1, [ks], ci - bi, mask=m_last)
            return 0

        lax.fori_loop(0, CHUNK // 16, grp, 0)
        pltpu.sync_copy(agbuf, ag_hbm.at[pl.ds(off, CHUNK)])
        pltpu.sync_copy(aibuf, ai_hbm.at[pl.ds(off, CHUNK)])
        return 0

    lax.fori_loop(0, NCHUNK, step, 0)
    pltpu.sync_copy(rsg1, rsg_hbm.at[cid, sid, 0])
    pltpu.sync_copy(rsi1, rsi_hbm.at[cid, sid, 0])


def _mask_sc(m_packed, h, t):
    mesh = plsc.VectorSubcoreMesh(core_axis_name="c", subcore_axis_name="s",
                                  num_cores=NC, num_subcores=NS)
    f = pl.kernel(
        _mask_sc_kernel,
        out_type=(
            jax.ShapeDtypeStruct((E_EDGES,), jnp.float32),
            jax.ShapeDtypeStruct((E_EDGES,), jnp.float32),
            jax.ShapeDtypeStruct((NC, NS, 1, NPAD), jnp.float32),
            jax.ShapeDtypeStruct((NC, NS, 1, NPAD), jnp.float32),
        ),
        mesh=mesh,
        scratch_types=[
            pltpu.VMEM((CHUNK,), jnp.int32),
            pltpu.VMEM((CHUNK,), jnp.int32),
            pltpu.VMEM((CHUNK, D), jnp.float32),
            pltpu.VMEM((CHUNK, D), jnp.float32),
            pltpu.VMEM((CHUNK,), jnp.float32),
            pltpu.VMEM((CHUNK,), jnp.float32),
            pltpu.VMEM((NPAD,), jnp.float32),
            pltpu.VMEM((NPAD,), jnp.float32),
            pltpu.SemaphoreType.DMA,
        ],
        compiler_params=pltpu.CompilerParams(needs_layout_passes=False),
    )
    ag, ai, rsg, rsi = f(m_packed, h, t)
    rs_g = jnp.sum(rsg.reshape(NW, NPAD), axis=0)[:N_NODES]
    rs_i = jnp.sum(rsi.reshape(NW, NPAD), axis=0)[:N_NODES]
    return ag, ai, rs_g, rs_i


def _intent_body(x_ref, w_ref, o_ref):
    x = x_ref[...]
    w = w_ref[...]
    logits = jnp.dot(x, w, preferred_element_type=jnp.float32)
    m = jnp.max(logits, axis=1, keepdims=True)
    p = jnp.exp(logits - m)
    s = jnp.sum(p, axis=1, keepdims=True)
    p = p / s
    o_ref[...] = lax.dot_general(p, w, (((1,), (1,)), ((), ())),
                                 preferred_element_type=jnp.float32)


def _intent(x, w):
    n = x.shape[0]
    grid = n // ROW_BLK
    return pl.pallas_call(
        _intent_body,
        grid=(grid,),
        in_specs=[
            pl.BlockSpec((ROW_BLK, D), lambda i: (i, 0)),
            pl.BlockSpec((D, N_INTENTS), lambda i: (0, 0)),
        ],
        out_specs=pl.BlockSpec((ROW_BLK, N_INTENTS), lambda i: (i, 0)),
        out_shape=jax.ShapeDtypeStruct((n, N_INTENTS), jnp.float32),
    )(x, w)


def _tc_b_body(gnn_ref, emb_ref, wu_ref, wi_ref, m_ref, int_ref):
    pid = pl.program_id(0)
    g = gnn_ref[...]
    n2 = jnp.sum(g * g, axis=1, keepdims=True)
    ng = g / jnp.maximum(jnp.sqrt(n2), 1e-12)
    x = emb_ref[...]
    w = jnp.where(pid < (N_USERS // ROW_BLK), wu_ref[...], wi_ref[...])
    logits = jnp.dot(x, w, preferred_element_type=jnp.float32)
    mx = jnp.max(logits, axis=1, keepdims=True)
    p = jnp.exp(logits - mx)
    p = p / jnp.sum(p, axis=1, keepdims=True)
    it = lax.dot_general(p, w, (((1,), (1,)), ((), ())),
                         preferred_element_type=jnp.float32)
    int_ref[...] = it
    n2i = jnp.sum(it * it, axis=1, keepdims=True)
    ni = it / jnp.maximum(jnp.sqrt(n2i), 1e-12)
    u_ng = lax.bitcast_convert_type(ng.astype(jnp.bfloat16),
                                    jnp.uint16).astype(jnp.uint32)
    u_ni = lax.bitcast_convert_type(ni.astype(jnp.bfloat16),
                                    jnp.uint16).astype(jnp.uint32)
    m_ref[...] = lax.bitcast_convert_type((u_ng << 16) | u_ni, jnp.float32)


def _tc_b(gnn, emb, wu, wi):
    blk = pl.BlockSpec((ROW_BLK, D), lambda i: (i, 0))
    wspec = pl.BlockSpec((D, N_INTENTS), lambda i: (0, 0))
    return pl.pallas_call(
        _tc_b_body,
        grid=(N_NODES // ROW_BLK,),
        in_specs=[blk, blk, wspec, wspec],
        out_specs=[blk, blk],
        out_shape=[jax.ShapeDtypeStruct((N_NODES, D), jnp.float32)] * 2,
    )(gnn, emb, wu, wi)


def _tc_e_body(gp_ref, ip_ref, dg_ref, di_ref, gnn_ref, int_ref, emb_ref,
               gaa_ref, iaa_ref, nxt_ref):
    gaa = dg_ref[...] * gp_ref[...]
    iaa = di_ref[...] * ip_ref[...]
    gaa_ref[...] = gaa
    iaa_ref[...] = iaa
    nxt_ref[...] = gnn_ref[...] + int_ref[...] + gaa + iaa + emb_ref[...]


def _tc_e(gaap, iaap, dg, di, gnn, int_e, emb):
    blk = pl.BlockSpec((ROW_BLK, D), lambda i: (i, 0))
    dspec = pl.BlockSpec((ROW_BLK, 1), lambda i: (i, 0))
    return pl.pallas_call(
        _tc_e_body,
        grid=(N_NODES // ROW_BLK,),
        in_specs=[blk, blk, dspec, dspec, blk, blk, blk],
        out_specs=[blk, blk, blk],
        out_shape=[jax.ShapeDtypeStruct((N_NODES, D), jnp.float32)] * 3,
    )(gaap, iaap, dg[:, None], di[:, None], gnn, int_e, emb)


def kernel(user_emb, item_emb, user_intent, item_intent, G_values, all_h, all_t):
    emb = jnp.concatenate([user_emb, item_emb], axis=0)
    gnn_l, int_l, gaa_l, iaa_l = [], [], [], []
    for i in range(N_LAYERS):
        gnn = _spmm_sc(G_values, all_h, all_t, emb)
        m_packed, int_e = _tc_b(gnn, emb, user_intent, item_intent)
        ag, ai, rsg, rsi = _mask_sc(m_packed, all_h, all_t)
        gaap = _spmm_sc(ag, all_h, all_t, emb)
        iaap = _spmm_sc(ai, all_h, all_t, emb)
        dg = jnp.where(rsg > 0, 1.0 / rsg, 0.0)
        di = jnp.where(rsi > 0, 1.0 / rsi, 0.0)
        gaa, iaa, emb_next = _tc_e(gaap, iaap, dg, di, gnn, int_e, emb)
        gnn_l.append(gnn)
        int_l.append(int_e)
        gaa_l.append(gaa)
        iaa_l.append(iaa)
        emb = emb_next
    return jnp.stack([jnp.stack(gnn_l), jnp.stack(int_l),
                      jnp.stack(gaa_l), jnp.stack(iaa_l)])


# speedup vs baseline: 9.0382x; 1.1359x over previous
import functools

import jax
import jax.numpy as jnp
from jax import lax
from jax.experimental import pallas as pl
from jax.experimental.pallas import tpu as pltpu
from jax.experimental.pallas import tpu_sc as plsc

N_USERS = 5000
N_ITEMS = 5000
N_NODES = N_USERS + N_ITEMS
E_EDGES = 320000
D = 128
N_INTENTS = 128
N_LAYERS = 2

ROW_BLK = 1000

NC = 2
NS = 16
NW = NC * NS
CHUNK = 80
ZROWS = N_NODES // NS
ZBLK = 125


def _spmm_kernel(vals_hbm, h_hbm, t_hbm, mat_hbm, out_hbm,
                 tb0, hb0, vb0, rw0, tb1, hb1, vb1, rw1, zbuf, acc, s0, s1):
    cid = lax.axis_index("c")
    sid = lax.axis_index("s")
    wid = sid * NC + cid
    tb, hb, vb, rw, sems = [tb0, tb1], [hb0, hb1], [vb0, vb1], [rw0, rw1], [s0, s1]

    def zrow(i, _):
        for k in range(D // 16):
            zbuf[i, pl.ds(k * 16, 16)] = jnp.zeros((16,), jnp.float32)
        return 0
    lax.fori_loop(0, ZBLK, zrow, 0)

    def zcopy(j, _):
        pltpu.sync_copy(zbuf, acc.at[pl.ds(sid * ZROWS + j * ZBLK, ZBLK)])
        return 0
    lax.fori_loop(0, ZROWS // ZBLK, zcopy, 0)
    plsc.subcore_barrier()

    base = wid * PERW

    def load_chunk(b, off):
        pltpu.sync_copy(t_hbm.at[pl.ds(off, CHUNK)], tb[b])
        pltpu.sync_copy(h_hbm.at[pl.ds(off, CHUNK)], hb[b])
        pltpu.sync_copy(vals_hbm.at[pl.ds(off, CHUNK)], vb[b])
        pltpu.async_copy(mat_hbm.at[tb[b]], rw[b], sems[b])

    def process(b):
        pltpu.make_async_copy(mat_hbm.at[tb[b]], rw[b], sems[b]).wait()
        rows = rw[b]

        def sgrp(g, _):
            vseg = vb[b][pl.ds(g * 16, 16)]
            for l in range(16):
                v = vseg[l]
                i = g * 16 + l
                for k in range(D // 16):
                    sl = pl.ds(k * 16, 16)
                    rows[i, sl] = rows[i, sl] * v
            return 0
        lax.fori_loop(0, CHUNK // 16, sgrp, 0)
        pltpu.sync_copy(rows, acc.at[hb[b]], add=True)

    load_chunk(0, base)

    def pair(p, _):
        c0 = 2 * p
        load_chunk(1, base + (c0 + 1) * CHUNK)
        process(0)

        @pl.when(c0 + 2 < NCHUNK)
        def _():
            load_chunk(0, base + (c0 + 2) * CHUNK)
        process(1)
        return 0
    lax.fori_loop(0, NCHUNK // 2, pair, 0)
    if NCHUNK % 2 == 1:
        process(0)
    plsc.subcore_barrier()

    pltpu.sync_copy(acc.at[pl.ds(sid * ZROWS, ZROWS)], out_hbm.at[cid, sid])


def _spmm_sc(vals, h, t, mat):
    mesh = plsc.VectorSubcoreMesh(core_axis_name="c", subcore_axis_name="s",
                                  num_cores=NC, num_subcores=NS)
    f = pl.kernel(
        _spmm_kernel,
        out_type=jax.ShapeDtypeStruct((NC, NS, ZROWS, D), jnp.float32),
        mesh=mesh,
        scratch_types=[
            pltpu.VMEM((CHUNK,), jnp.int32),
            pltpu.VMEM((CHUNK,), jnp.int32),
            pltpu.VMEM((CHUNK,), jnp.float32),
            pltpu.VMEM((CHUNK, D), jnp.float32),
            pltpu.VMEM((CHUNK,), jnp.int32),
            pltpu.VMEM((CHUNK,), jnp.int32),
            pltpu.VMEM((CHUNK,), jnp.float32),
            pltpu.VMEM((CHUNK, D), jnp.float32),
            pltpu.VMEM((ZBLK, D), jnp.float32),
            pltpu.VMEM_SHARED((N_NODES, D), jnp.float32),
            pltpu.SemaphoreType.DMA,
            pltpu.SemaphoreType.DMA,
        ],
    )
    part = f(vals, h, t, mat)
    part = part.reshape(NC, N_NODES, D)
    return part[0] + part[1]


NPAD = 10240
PERW = E_EDGES // NW
NCHUNK = PERW // CHUNK


def _dyn16(v, idx):
    dnums = lax.GatherDimensionNumbers(
        offset_dims=(), collapsed_slice_dims=(0,), start_index_map=(0,))
    return lax.gather(v, idx[:, None], dnums, (1,),
                      mode=lax.GatherScatterMode.PROMISE_IN_BOUNDS)


def _mask_sc_kernel(m_hbm, h_hbm, t_hbm, ag_hbm, ai_hbm,
                    rsg_hbm, rsi_hbm,
                    hbuf0, tbuf0, mh0, mt0, hbuf1, tbuf1, mh1, mt1,
                    agbuf, aibuf, rsg1, rsi1, s0, s1):
    cid = lax.axis_index("c")
    sid = lax.axis_index("s")
    wid = sid * NC + cid
    hbl, tbl = [hbuf0, hbuf1], [tbuf0, tbuf1]
    mhl, mtl, sems = [mh0, mh1], [mt0, mt1], [s0, s1]
    lanes = lax.iota(jnp.int32, 16)
    himask = jnp.full((16,), -65536, jnp.int32)
    perms = [lanes ^ sh for sh in (8, 4, 2, 1)]
    cols = [k * 16 + lanes for k in range(D // 16)]

    def hsum(v):
        for p in perms:
            v = v + _dyn16(v, p)
        return v

    def zrow(i, _):
        sl = pl.ds(i * 16, 16)
        rsg1[sl] = jnp.zeros((16,), jnp.float32)
        rsi1[sl] = jnp.zeros((16,), jnp.float32)
        return 0
    lax.fori_loop(0, NPAD // 16, zrow, 0)

    base = wid * PERW

    def load_chunk(b, off):
        pltpu.sync_copy(h_hbm.at[pl.ds(off, CHUNK)], hbl[b])
        pltpu.sync_copy(t_hbm.at[pl.ds(off, CHUNK)], tbl[b])
        pltpu.async_copy(m_hbm.at[hbl[b]], mhl[b], sems[b])
        pltpu.async_copy(m_hbm.at[tbl[b]], mtl[b], sems[b])

    def process(b, off):
        pltpu.make_async_copy(m_hbm.at[hbl[b]], mhl[b], sems[b]).wait()
        pltpu.make_async_copy(m_hbm.at[tbl[b]], mtl[b], sems[b]).wait()
        hbuf, mh, mt = hbl[b], mhl[b], mtl[b]

        def grp(g, _):
            row0 = g * 16
            alpha_g = jnp.zeros((16,), jnp.float32)
            alpha_i = jnp.zeros((16,), jnp.float32)
            for l in range(16):
                row = row0 + l
                rsplat = jnp.full((16,), row, jnp.int32)
                accg = jnp.zeros((16,), jnp.float32)
                acci = jnp.zeros((16,), jnp.float32)
                for k in range(D // 16):
                    bh = plsc.bitcast(
                        plsc.load_gather(mh, [rsplat, cols[k]]), jnp.int32)
                    bt = plsc.bitcast(
                        plsc.load_gather(mt, [rsplat, cols[k]]), jnp.int32)
                    hg = plsc.bitcast(bh & himask, jnp.float32)
                    tg = plsc.bitcast(bt & himask, jnp.float32)
                    hi = plsc.bitcast(bh << 16, jnp.float32)
                    ti = plsc.bitcast(bt << 16, jnp.float32)
                    accg = accg + hg * tg
                    acci = acci + hi * ti
                lmask = lanes == l
                alpha_g = jnp.where(lmask, (hsum(accg) + 1.0) * 0.5, alpha_g)
                alpha_i = jnp.where(lmask, (hsum(acci) + 1.0) * 0.5, alpha_i)
            agbuf[pl.ds(row0, 16)] = alpha_g
            aibuf[pl.ds(row0, 16)] = alpha_i

            hv = hbuf[pl.ds(row0, 16)]
            ks, perm = plsc.sort_key_val(hv, lanes)
            ag_s = _dyn16(alpha_g, perm)
            ai_s = _dyn16(alpha_i, perm)
            cg = plsc.cumsum(ag_s)
            ci = plsc.cumsum(ai_s)
            eg = cg - ag_s
            ei = ci - ai_s
            kp = _dyn16(ks, jnp.maximum(lanes - 1, 0))
            m_first = jnp.logical_or(jnp.not_equal(ks, kp), lanes == 0)
            zero = jnp.zeros((16,), jnp.float32)
            bg = plsc.cummax(jnp.where(m_first, eg, zero))
            bi = plsc.cummax(jnp.where(m_first, ei, zero))
            kn = _dyn16(ks, jnp.minimum(lanes + 1, 15))
            m_last = jnp.logical_or(jnp.not_equal(ks, kn), lanes == 15)
            plsc.addupdate_scatter(rsg1, [ks], cg - bg, mask=m_last)
            plsc.addupdate_scatter(rsi1, [ks], ci - bi, mask=m_last)
            return 0

        lax.fori_loop(0, CHUNK // 16, grp, 0)
        pltpu.sync_copy(agbuf, ag_hbm.at[pl.ds(off, CHUNK)])
        pltpu.sync_copy(aibuf, ai_hbm.at[pl.ds(off, CHUNK)])
        return 0

    load_chunk(0, base)

    def pair(p, _):
        c0 = 2 * p
        load_chunk(1, base + (c0 + 1) * CHUNK)
        process(0, base + c0 * CHUNK)

        @pl.when(c0 + 2 < NCHUNK)
        def _():
            load_chunk(0, base + (c0 + 2) * CHUNK)
        process(1, base + (c0 + 1) * CHUNK)
        return 0
    lax.fori_loop(0, NCHUNK // 2, pair, 0)
    if NCHUNK % 2 == 1:
        process(0, base + (NCHUNK - 1) * CHUNK)
    pltpu.sync_copy(rsg1, rsg_hbm.at[cid, sid, 0])
    pltpu.sync_copy(rsi1, rsi_hbm.at[cid, sid, 0])


def _mask_sc(m_packed, h, t):
    mesh = plsc.VectorSubcoreMesh(core_axis_name="c", subcore_axis_name="s",
                                  num_cores=NC, num_subcores=NS)
    f = pl.kernel(
        _mask_sc_kernel,
        out_type=(
            jax.ShapeDtypeStruct((E_EDGES,), jnp.float32),
            jax.ShapeDtypeStruct((E_EDGES,), jnp.float32),
            jax.ShapeDtypeStruct((NC, NS, 1, NPAD), jnp.float32),
            jax.ShapeDtypeStruct((NC, NS, 1, NPAD), jnp.float32),
        ),
        mesh=mesh,
        scratch_types=[
            pltpu.VMEM((CHUNK,), jnp.int32),
            pltpu.VMEM((CHUNK,), jnp.int32),
            pltpu.VMEM((CHUNK, D), jnp.float32),
            pltpu.VMEM((CHUNK, D), jnp.float32),
            pltpu.VMEM((CHUNK,), jnp.int32),
            pltpu.VMEM((CHUNK,), jnp.int32),
            pltpu.VMEM((CHUNK, D), jnp.float32),
            pltpu.VMEM((CHUNK, D), jnp.float32),
            pltpu.VMEM((CHUNK,), jnp.float32),
            pltpu.VMEM((CHUNK,), jnp.float32),
            pltpu.VMEM((NPAD,), jnp.float32),
            pltpu.VMEM((NPAD,), jnp.float32),
            pltpu.SemaphoreType.DMA,
            pltpu.SemaphoreType.DMA,
        ],
        compiler_params=pltpu.CompilerParams(needs_layout_passes=False),
    )
    ag, ai, rsg, rsi = f(m_packed, h, t)
    rs_g = jnp.sum(rsg.reshape(NW, NPAD), axis=0)[:N_NODES]
    rs_i = jnp.sum(rsi.reshape(NW, NPAD), axis=0)[:N_NODES]
    return ag, ai, rs_g, rs_i


def _intent_body(x_ref, w_ref, o_ref):
    x = x_ref[...]
    w = w_ref[...]
    logits = jnp.dot(x, w, preferred_element_type=jnp.float32)
    m = jnp.max(logits, axis=1, keepdims=True)
    p = jnp.exp(logits - m)
    s = jnp.sum(p, axis=1, keepdims=True)
    p = p / s
    o_ref[...] = lax.dot_general(p, w, (((1,), (1,)), ((), ())),
                                 preferred_element_type=jnp.float32)


def _intent(x, w):
    n = x.shape[0]
    grid = n // ROW_BLK
    return pl.pallas_call(
        _intent_body,
        grid=(grid,),
        in_specs=[
            pl.BlockSpec((ROW_BLK, D), lambda i: (i, 0)),
            pl.BlockSpec((D, N_INTENTS), lambda i: (0, 0)),
        ],
        out_specs=pl.BlockSpec((ROW_BLK, N_INTENTS), lambda i: (i, 0)),
        out_shape=jax.ShapeDtypeStruct((n, N_INTENTS), jnp.float32),
    )(x, w)


def _tc_b_body(gnn_ref, emb_ref, wu_ref, wi_ref, m_ref, int_ref):
    pid = pl.program_id(0)
    g = gnn_ref[...]
    n2 = jnp.sum(g * g, axis=1, keepdims=True)
    ng = g / jnp.maximum(jnp.sqrt(n2), 1e-12)
    x = emb_ref[...]
    w = jnp.where(pid < (N_USERS // ROW_BLK), wu_ref[...], wi_ref[...])
    logits = jnp.dot(x, w, preferred_element_type=jnp.float32)
    mx = jnp.max(logits, axis=1, keepdims=True)
    p = jnp.exp(logits - mx)
    p = p / jnp.sum(p, axis=1, keepdims=True)
    it = lax.dot_general(p, w, (((1,), (1,)), ((), ())),
                         preferred_element_type=jnp.float32)
    int_ref[...] = it
    n2i = jnp.sum(it * it, axis=1, keepdims=True)
    ni = it / jnp.maximum(jnp.sqrt(n2i), 1e-12)
    u_ng = lax.bitcast_convert_type(ng.astype(jnp.bfloat16),
                                    jnp.uint16).astype(jnp.uint32)
    u_ni = lax.bitcast_convert_type(ni.astype(jnp.bfloat16),
                                    jnp.uint16).astype(jnp.uint32)
    m_ref[...] = lax.bitcast_convert_type((u_ng << 16) | u_ni, jnp.float32)


def _tc_b(gnn, emb, wu, wi):
    blk = pl.BlockSpec((ROW_BLK, D), lambda i: (i, 0))
    wspec = pl.BlockSpec((D, N_INTENTS), lambda i: (0, 0))
    return pl.pallas_call(
        _tc_b_body,
        grid=(N_NODES // ROW_BLK,),
        in_specs=[blk, blk, wspec, wspec],
        out_specs=[blk, blk],
        out_shape=[jax.ShapeDtypeStruct((N_NODES, D), jnp.float32)] * 2,
    )(gnn, emb, wu, wi)


def _tc_e_body(gp_ref, ip_ref, dg_ref, di_ref, gnn_ref, int_ref, emb_ref,
               gaa_ref, iaa_ref, nxt_ref):
    gaa = dg_ref[...] * gp_ref[...]
    iaa = di_ref[...] * ip_ref[...]
    gaa_ref[...] = gaa
    iaa_ref[...] = iaa
    nxt_ref[...] = gnn_ref[...] + int_ref[...] + gaa + iaa + emb_ref[...]


def _tc_e(gaap, iaap, dg, di, gnn, int_e, emb):
    blk = pl.BlockSpec((ROW_BLK, D), lambda i: (i, 0))
    dspec = pl.BlockSpec((ROW_BLK, 1), lambda i: (i, 0))
    return pl.pallas_call(
        _tc_e_body,
        grid=(N_NODES // ROW_BLK,),
        in_specs=[blk, blk, dspec, dspec, blk, blk, blk],
        out_specs=[blk, blk, blk],
        out_shape=[jax.ShapeDtypeStruct((N_NODES, D), jnp.float32)] * 3,
    )(gaap, iaap, dg[:, None], di[:, None], gnn, int_e, emb)


def kernel(user_emb, item_emb, user_intent, item_intent, G_values, all_h, all_t):
    emb = jnp.concatenate([user_emb, item_emb], axis=0)
    gnn_l, int_l, gaa_l, iaa_l = [], [], [], []
    for i in range(N_LAYERS):
        gnn = _spmm_sc(G_values, all_h, all_t, emb)
        m_packed, int_e = _tc_b(gnn, emb, user_intent, item_intent)
        ag, ai, rsg, rsi = _mask_sc(m_packed, all_h, all_t)
        gaap = _spmm_sc(ag, all_h, all_t, emb)
        iaap = _spmm_sc(ai, all_h, all_t, emb)
        dg = jnp.where(rsg > 0, 1.0 / rsg, 0.0)
        di = jnp.where(rsi > 0, 1.0 / rsi, 0.0)
        gaa, iaa, emb_next = _tc_e(gaap, iaap, dg, di, gnn, int_e, emb)
        gnn_l.append(gnn)
        int_l.append(int_e)
        gaa_l.append(gaa)
        iaa_l.append(iaa)
        emb = emb_next
    return jnp.stack([jnp.stack(gnn_l), jnp.stack(int_l),
                      jnp.stack(gaa_l), jnp.stack(iaa_l)])
